# baseline (speedup 1.0000x reference)
import functools

import jax
import jax.numpy as jnp
from jax import lax
from jax.experimental import pallas as pl
from jax.experimental.pallas import tpu as pltpu
from jax.experimental.pallas import tpu_sc as plsc

N = 10000
E = 320000
D = 128
DOUT = 40
D2 = 64
NPAD = 10240
NC = 2
NS = 16
CHUNK = 80
ROWS_PER_TILE = NPAD // NS
EX = 332800

_MESH = dict(core_axis_name="c", subcore_axis_name="s")


def _fast_rsqrt(v):
    i = plsc.bitcast(v, jnp.int32)
    i = jnp.int32(0x5F3759DF) - lax.shift_right_logical(i, 1)
    y = plsc.bitcast(i, jnp.float32)
    for _ in range(3):
        y = y * (1.5 - 0.5 * v * y * y)
    return y


def _sc_degrees_norms_xs(edgesx, x):
    nblk = EX // (NC * NS * CHUNK)

    @functools.partial(
        pl.kernel,
        out_type=[
            jax.ShapeDtypeStruct((NC * NPAD,), jnp.float32),
            jax.ShapeDtypeStruct((NPAD, D), jnp.float32),
        ],
        mesh=plsc.VectorSubcoreMesh(**_MESH),
        compiler_params=pltpu.CompilerParams(needs_layout_passes=False),
        scratch_types=[
            pltpu.VMEM((2, nblk, CHUNK), jnp.int32),
            pltpu.VMEM((CHUNK,), jnp.float32),
            pltpu.VMEM((ROWS_PER_TILE,), jnp.float32),
            pltpu.VMEM((2, 160, D), jnp.float32),
            pltpu.SemaphoreType.DMA,
            pltpu.SemaphoreType.DMA,
            pltpu.SemaphoreType.DMA,
            pltpu.VMEM_SHARED((NPAD,), jnp.float32),
        ],
    )
    def deg_kernel(edges_hbm, x_hbm, norm_hbm, xs_hbm, idx_all, ones_v,
                   nbuf, xio, ssem, xisem, xosem, deg_s):
        c = lax.axis_index("c")
        s = lax.axis_index("s")
        w = c * NS + s

        def fill_zero(i, carry):
            nbuf[pl.ds(i * 16, 16)] = jnp.zeros((16,), jnp.float32)
            return carry

        lax.fori_loop(0, ROWS_PER_TILE // 16, fill_zero, 0)

        def fill_one(i, carry):
            ones_v[pl.ds(i * 16, 16)] = jnp.ones((16,), jnp.float32)
            return carry

        lax.fori_loop(0, CHUNK // 16, fill_one, 0)

        pltpu.sync_copy(edges_hbm.at[2 * w], idx_all.at[0])
        pltpu.sync_copy(edges_hbm.at[2 * w + 1], idx_all.at[1])
        pltpu.sync_copy(nbuf, deg_s.at[pl.ds(s * ROWS_PER_TILE, ROWS_PER_TILE)])
        plsc.subcore_barrier()

        for h in range(2):
            def step(j, carry):
                pltpu.async_copy(ones_v, deg_s.at[idx_all.at[h, j]], ssem, add=True)
                return carry

            lax.fori_loop(0, nblk, step, 0)

        def drain(j, carry):
            pltpu.make_async_copy(ones_v, deg_s.at[idx_all.at[0, 0]], ssem).wait()
            return carry

        lax.fori_loop(0, 2 * nblk, drain, 0)
        plsc.subcore_barrier()

        pltpu.sync_copy(deg_s.at[pl.ds(s * ROWS_PER_TILE, ROWS_PER_TILE)], nbuf)

        def rsq(i, carry):
            nbuf[pl.ds(i * 16, 16)] = _fast_rsqrt(nbuf[pl.ds(i * 16, 16)])
            return carry

        lax.fori_loop(0, ROWS_PER_TILE // 16, rsq, 0)
        pltpu.sync_copy(
            nbuf, norm_hbm.at[pl.ds(c * NPAD + s * ROWS_PER_TILE, ROWS_PER_TILE)]
        )

        def scale_blocks(nb2, BR):
            base = s * ROWS_PER_TILE

            def wait_in():
                pltpu.make_async_copy(
                    x_hbm.at[pl.ds(0, BR)], xio.at[0, pl.ds(0, BR)], xisem
                ).wait()

            def wait_out():
                pltpu.make_async_copy(
                    xio.at[0, pl.ds(0, BR)], xs_hbm.at[pl.ds(0, BR)], xosem
                ).wait()

            pltpu.async_copy(
                x_hbm.at[pl.ds(base, BR)], xio.at[0, pl.ds(0, BR)], xisem
            )
            for b in range(nb2):
                bb = b % 2
                if b >= 1:
                    wait_out()
                if b + 1 < nb2:
                    pltpu.async_copy(
                        x_hbm.at[pl.ds(base + (b + 1) * BR, BR)],
                        xio.at[(b + 1) % 2, pl.ds(0, BR)], xisem,
                    )
                wait_in()

                def rowgroup(g, carry):
                    nv = nbuf[pl.ds(b * BR + g * 16, 16)]
                    for j in range(16):
                        cfv = nv.at[jnp.full((16,), j, jnp.int32)].get(
                            mode="promise_in_bounds"
                        )
                        r = g * 16 + j
                        for vv in range(D // 16):
                            xio[bb, r, pl.ds(vv * 16, 16)] = (
                                xio[bb, r, pl.ds(vv * 16, 16)] * cfv
                            )
                    return carry

                lax.fori_loop(0, BR // 16, rowgroup, 0)
                pltpu.async_copy(
                    xio.at[bb, pl.ds(0, BR)],
                    xs_hbm.at[pl.ds(base + b * BR, BR)], xosem,
                )
            wait_out()

        @pl.when(jnp.logical_and(c == 0, s < NS - 1))
        def _():
            scale_blocks(4, 160)

        @pl.when(jnp.logical_and(c == 0, s == NS - 1))
        def _():
            scale_blocks(5, 80)

    return deg_kernel(edgesx, x)


def _sc_scatter(edgesx, feat, dfeat):
    ept = EX // (NC * NS)
    csz = CHUNK
    NB = 3 if dfeat > 64 else 4
    nchunk = ept // csz
    NI = NB + 2
    ZR = 16

    @functools.partial(
        pl.kernel,
        out_type=jax.ShapeDtypeStruct((NC * NPAD, dfeat), jnp.float32),
        mesh=plsc.VectorSubcoreMesh(**_MESH),
        compiler_params=pltpu.CompilerParams(
            use_tc_tiling_on_sc=(dfeat % 128 == 0)
        ),
        scratch_types=[
            pltpu.VMEM((NI, csz), jnp.int32),
            pltpu.VMEM((NI, csz), jnp.int32),
            pltpu.VMEM((NB, csz, dfeat), jnp.float32),
            pltpu.VMEM((ZR, dfeat), jnp.float32),
            pltpu.SemaphoreType.DMA,
            pltpu.SemaphoreType.DMA,
            pltpu.SemaphoreType.DMA,
            pltpu.VMEM_SHARED((NPAD, dfeat), jnp.float32),
        ],
    )
    def scat_kernel(edges_hbm, feat_hbm, out_hbm, sidx, didx,
                    rows, zrows, isem, gsem, ssem, agg_s):
        c = lax.axis_index("c")
        s = lax.axis_index("s")
        wid = c * NS + s
        vpr = dfeat // 16

        def fill_zero(k, carry):
            zrows[k // vpr, pl.ds((k % vpr) * 16, 16)] = jnp.zeros((16,), jnp.float32)
            return carry

        lax.fori_loop(0, ZR * vpr, fill_zero, 0)

        def zero_chunk(k, carry):
            pltpu.sync_copy(
                zrows, agg_s.at[pl.ds(s * ROWS_PER_TILE + k * ZR, ZR)]
            )
            return carry

        lax.fori_loop(0, ROWS_PER_TILE // ZR, zero_chunk, 0)
        plsc.subcore_barrier()

        def load_idx(j, slot):
            pltpu.async_copy(edges_hbm.at[wid, j], sidx.at[slot], isem)
            pltpu.async_copy(edges_hbm.at[NC * NS + wid, j], didx.at[slot], isem)

        def wait_idx():
            pltpu.make_async_copy(edges_hbm.at[0, 0], sidx.at[0], isem).wait()
            pltpu.make_async_copy(edges_hbm.at[0, 0], didx.at[0], isem).wait()

        def gath(b, slot):
            pltpu.async_copy(feat_hbm.at[sidx.at[slot]], rows.at[b], gsem)

        def wait_gath():
            pltpu.make_async_copy(
                feat_hbm.at[sidx.at[0]], rows.at[0], gsem
            ).wait()

        def scat(b, slot):
            pltpu.async_copy(rows.at[b], agg_s.at[didx.at[slot]], ssem, add=True)

        def wait_scat():
            pltpu.make_async_copy(
                rows.at[0], agg_s.at[didx.at[0]], ssem
            ).wait()

        n = nchunk
        for j in range(NI):
            load_idx(j, j)
        for j in range(NB - 1):
            wait_idx()
            gath(j, j)
        wait_gath()
        scat(0, 0)
        wait_idx()
        gath((NB - 1) % NB, (NB - 1) % NI)
        wait_gath()
        scat(1 % NB, 1 % NI)

        def step(k, carry):
            wait_scat()
            load_idx(k + NI - 2, lax.rem(k + NI - 2, NI))
            wait_idx()
            gath(lax.rem(k + NB - 2, NB), lax.rem(k + NB - 2, NI))
            wait_gath()
            scat(lax.rem(k, NB), lax.rem(k, NI))
            return carry

        lax.fori_loop(2, n - NI + 2, step, 0)

        for k in range(n - NI + 2, n - NB + 2):
            wait_scat()
            wait_idx()
            gath((k + NB - 2) % NB, (k + NB - 2) % NI)
            wait_gath()
            scat(k % NB, k % NI)
        for k in range(n - NB + 2, n):
            wait_scat()
            wait_gath()
            scat(k % NB, k % NI)
        wait_scat()
        wait_scat()

        plsc.subcore_barrier()
        pltpu.sync_copy(
            agg_s.at[pl.ds(s * ROWS_PER_TILE, ROWS_PER_TILE)],
            out_hbm.at[pl.ds(c * NPAD + s * ROWS_PER_TILE, ROWS_PER_TILE)],
        )

    return scat_kernel(edgesx, feat)


_TCR = 1280


def _mm_body(a0_ref, a1_ref, c2_ref, ns_ref, w1_ref, b1_ref, w2_ref, g2_ref):
    hp = lax.Precision.HIGHEST
    w12 = jnp.dot(w1_ref[...], w2_ref[...],
                  preferred_element_type=jnp.float32, precision=hp)
    v = jnp.dot(b1_ref[...], w2_ref[...],
                preferred_element_type=jnp.float32, precision=hp)
    a = (a0_ref[...] + a1_ref[...]) * c2_ref[...]
    g2_ref[...] = (
        jnp.dot(a, w12, preferred_element_type=jnp.float32, precision=hp)
        + ns_ref[...] * v
    )


def _tc_matmuls(a0, a1, c2_col, ns_col, W1, b1r, W2p):
    row = lambda i: (i, 0)
    full = lambda i: (0, 0)
    return pl.pallas_call(
        _mm_body,
        grid=(NPAD // _TCR,),
        in_specs=[
            pl.BlockSpec((_TCR, D), row),
            pl.BlockSpec((_TCR, D), row),
            pl.BlockSpec((_TCR, 1), row),
            pl.BlockSpec((_TCR, 1), row),
            pl.BlockSpec((D, D), full),
            pl.BlockSpec((1, D), full),
            pl.BlockSpec((D, D2), full),
        ],
        out_specs=pl.BlockSpec((_TCR, D2), row),
        out_shape=jax.ShapeDtypeStruct((NPAD, D2), jnp.float32),
    )(a0, a1, c2_col, ns_col, W1, b1r, W2p)


_TCF = 2000


def _final_body(a0_ref, a1_ref, nd_ref, b2_ref, out_ref):
    out_ref[...] = (a0_ref[...] + a1_ref[...]) * nd_ref[...] + b2_ref[...]


def _tc_final(a0, a1, nd_col, b2r):
    row = lambda i: (i, 0)
    full = lambda i: (0, 0)
    return pl.pallas_call(
        _final_body,
        grid=(N // _TCF,),
        in_specs=[
            pl.BlockSpec((_TCF, D2), row),
            pl.BlockSpec((_TCF, D2), row),
            pl.BlockSpec((_TCF, 1), row),
            pl.BlockSpec((1, D2), full),
        ],
        out_specs=pl.BlockSpec((_TCF, D2), row),
        out_shape=jax.ShapeDtypeStruct((N, D2), jnp.float32),
    )(a0, a1, nd_col, b2r)


def kernel(x, edge_index, W1, b1, W2, b2):
    npadedge = EX - E - N
    loops = jnp.arange(N, dtype=jnp.int32)
    junk = jnp.full((npadedge,), NPAD - 1, jnp.int32)
    srcx = jnp.concatenate([edge_index[0], loops, junk])
    dstx = jnp.concatenate([edge_index[1], loops, junk])
    edgesx = jnp.concatenate([srcx, dstx]).reshape(
        2 * NC * NS, EX // (NC * NS * CHUNK), CHUNK
    )

    norms, xs = _sc_degrees_norms_xs(edgesx, x)

    agg = _sc_scatter(edgesx, xs, D)
    c2_col = (norms[:NPAD] * norms[NPAD:])[:, None]
    g2 = _tc_matmuls(
        agg[:NPAD], agg[NPAD:], c2_col, norms[:NPAD, None],
        W1, b1[None, :], jnp.pad(W2, ((0, 0), (0, D2 - DOUT))),
    )

    agg2 = _sc_scatter(edgesx, g2, D2)
    out = _tc_final(
        agg2[:N], agg2[NPAD:NPAD + N], norms[NPAD:NPAD + N, None],
        jnp.pad(b2, (0, D2 - DOUT))[None, :],
    )
    return out[:, :DOUT]


# speedup vs baseline: 1.6211x; 1.6211x over previous
import functools

import jax
import jax.numpy as jnp
from jax import lax
from jax.experimental import pallas as pl
from jax.experimental.pallas import tpu as pltpu
from jax.experimental.pallas import tpu_sc as plsc

N = 10000
E = 320000
D = 128
DOUT = 40
D2 = 64
NPAD = 10240
NC = 2
NS = 16
CHUNK = 80
ROWS_PER_TILE = NPAD // NS
EX = 332800

_MESH = dict(core_axis_name="c", subcore_axis_name="s")


def _fast_rsqrt(v):
    i = plsc.bitcast(v, jnp.int32)
    i = jnp.int32(0x5F3759DF) - lax.shift_right_logical(i, 1)
    y = plsc.bitcast(i, jnp.float32)
    for _ in range(3):
        y = y * (1.5 - 0.5 * v * y * y)
    return y


def _sc_degrees_norms_xs(edgesx, x):
    nblk = EX // (NC * NS * CHUNK)

    @functools.partial(
        pl.kernel,
        out_type=[
            jax.ShapeDtypeStruct((NC * NPAD,), jnp.float32),
            jax.ShapeDtypeStruct((NPAD, D), jnp.float32),
        ],
        mesh=plsc.VectorSubcoreMesh(**_MESH),
        compiler_params=pltpu.CompilerParams(needs_layout_passes=False),
        scratch_types=[
            pltpu.VMEM((2, nblk, CHUNK), jnp.int32),
            pltpu.VMEM((CHUNK,), jnp.float32),
            pltpu.VMEM((ROWS_PER_TILE,), jnp.float32),
            pltpu.VMEM((2, 160, D), jnp.float32),
            pltpu.SemaphoreType.DMA,
            pltpu.SemaphoreType.DMA,
            pltpu.SemaphoreType.DMA,
            pltpu.VMEM_SHARED((NPAD,), jnp.float32),
        ],
    )
    def deg_kernel(edges_hbm, x_hbm, norm_hbm, xs_hbm, idx_all, ones_v,
                   nbuf, xio, ssem, xisem, xosem, deg_s):
        c = lax.axis_index("c")
        s = lax.axis_index("s")
        w = c * NS + s

        def fill_zero(i, carry):
            nbuf[pl.ds(i * 16, 16)] = jnp.zeros((16,), jnp.float32)
            return carry

        lax.fori_loop(0, ROWS_PER_TILE // 16, fill_zero, 0)

        def fill_one(i, carry):
            ones_v[pl.ds(i * 16, 16)] = jnp.ones((16,), jnp.float32)
            return carry

        lax.fori_loop(0, CHUNK // 16, fill_one, 0)

        pltpu.sync_copy(edges_hbm.at[2 * w], idx_all.at[0])
        pltpu.sync_copy(edges_hbm.at[2 * w + 1], idx_all.at[1])
        pltpu.sync_copy(nbuf, deg_s.at[pl.ds(s * ROWS_PER_TILE, ROWS_PER_TILE)])
        plsc.subcore_barrier()

        for h in range(2):
            def step(j, carry):
                pltpu.async_copy(ones_v, deg_s.at[idx_all.at[h, j]], ssem, add=True)
                return carry

            lax.fori_loop(0, nblk, step, 0)

        def drain(j, carry):
            pltpu.make_async_copy(ones_v, deg_s.at[idx_all.at[0, 0]], ssem).wait()
            return carry

        lax.fori_loop(0, 2 * nblk, drain, 0)
        plsc.subcore_barrier()

        pltpu.sync_copy(deg_s.at[pl.ds(s * ROWS_PER_TILE, ROWS_PER_TILE)], nbuf)

        def rsq(i, carry):
            nbuf[pl.ds(i * 16, 16)] = _fast_rsqrt(nbuf[pl.ds(i * 16, 16)])
            return carry

        lax.fori_loop(0, ROWS_PER_TILE // 16, rsq, 0)
        pltpu.sync_copy(
            nbuf, norm_hbm.at[pl.ds(c * NPAD + s * ROWS_PER_TILE, ROWS_PER_TILE)]
        )

        def scale_blocks(nb2, BR):
            base = s * ROWS_PER_TILE

            def wait_in():
                pltpu.make_async_copy(
                    x_hbm.at[pl.ds(0, BR)], xio.at[0, pl.ds(0, BR)], xisem
                ).wait()

            def wait_out():
                pltpu.make_async_copy(
                    xio.at[0, pl.ds(0, BR)], xs_hbm.at[pl.ds(0, BR)], xosem
                ).wait()

            pltpu.async_copy(
                x_hbm.at[pl.ds(base, BR)], xio.at[0, pl.ds(0, BR)], xisem
            )
            for b in range(nb2):
                bb = b % 2
                if b >= 1:
                    wait_out()
                if b + 1 < nb2:
                    pltpu.async_copy(
                        x_hbm.at[pl.ds(base + (b + 1) * BR, BR)],
                        xio.at[(b + 1) % 2, pl.ds(0, BR)], xisem,
                    )
                wait_in()

                def rowgroup(g, carry):
                    nv = nbuf[pl.ds(b * BR + g * 16, 16)]
                    for j in range(16):
                        cfv = nv.at[jnp.full((16,), j, jnp.int32)].get(
                            mode="promise_in_bounds"
                        )
                        r = g * 16 + j
                        for vv in range(D // 16):
                            xio[bb, r, pl.ds(vv * 16, 16)] = (
                                xio[bb, r, pl.ds(vv * 16, 16)] * cfv
                            )
                    return carry

                lax.fori_loop(0, BR // 16, rowgroup, 0)
                pltpu.async_copy(
                    xio.at[bb, pl.ds(0, BR)],
                    xs_hbm.at[pl.ds(base + b * BR, BR)], xosem,
                )
            wait_out()

        @pl.when(jnp.logical_and(c == 0, s < NS - 1))
        def _():
            scale_blocks(4, 160)

        @pl.when(jnp.logical_and(c == 0, s == NS - 1))
        def _():
            scale_blocks(5, 80)

    return deg_kernel(edgesx, x)


def _sc_scatter(edgesx, feat, dfeat):
    ept = EX // (NC * NS)
    csz = CHUNK
    NB = 3 if dfeat > 64 else 4
    nchunk = ept // csz
    NI = NB + 2
    ZR = 16

    @functools.partial(
        pl.kernel,
        out_type=jax.ShapeDtypeStruct((NC * NPAD, dfeat), jnp.float32),
        mesh=plsc.VectorSubcoreMesh(**_MESH),
        compiler_params=pltpu.CompilerParams(
            use_tc_tiling_on_sc=(dfeat % 128 == 0)
        ),
        scratch_types=[
            pltpu.VMEM((NI, csz), jnp.int32),
            pltpu.VMEM((NI, csz), jnp.int32),
            pltpu.VMEM((NB, csz, dfeat), jnp.float32),
            pltpu.VMEM((ZR, dfeat), jnp.float32),
            pltpu.SemaphoreType.DMA,
            pltpu.SemaphoreType.DMA,
            pltpu.SemaphoreType.DMA,
            pltpu.VMEM_SHARED((NPAD, dfeat), jnp.float32),
        ],
    )
    def scat_kernel(edges_hbm, feat_hbm, out_hbm, sidx, didx,
                    rows, zrows, isem, gsem, ssem, agg_s):
        c = lax.axis_index("c")
        s = lax.axis_index("s")
        wid = c * NS + s
        vpr = dfeat // 16

        def fill_zero(k, carry):
            zrows[k // vpr, pl.ds((k % vpr) * 16, 16)] = jnp.zeros((16,), jnp.float32)
            return carry

        lax.fori_loop(0, ZR * vpr, fill_zero, 0)

        def zero_chunk(k, carry):
            pltpu.sync_copy(
                zrows, agg_s.at[pl.ds(s * ROWS_PER_TILE + k * ZR, ZR)]
            )
            return carry

        lax.fori_loop(0, ROWS_PER_TILE // ZR, zero_chunk, 0)
        plsc.subcore_barrier()

        def load_idx(j, slot):
            pltpu.async_copy(edges_hbm.at[wid, j], sidx.at[slot], isem)
            pltpu.async_copy(edges_hbm.at[NC * NS + wid, j], didx.at[slot], isem)

        def wait_idx():
            pltpu.make_async_copy(edges_hbm.at[0, 0], sidx.at[0], isem).wait()
            pltpu.make_async_copy(edges_hbm.at[0, 0], didx.at[0], isem).wait()

        def gath(b, slot):
            pltpu.async_copy(feat_hbm.at[sidx.at[slot]], rows.at[b], gsem)

        def wait_gath():
            pltpu.make_async_copy(
                feat_hbm.at[sidx.at[0]], rows.at[0], gsem
            ).wait()

        def scat(b, slot):
            pltpu.async_copy(rows.at[b], agg_s.at[didx.at[slot]], ssem, add=True)

        def wait_scat():
            pltpu.make_async_copy(
                rows.at[0], agg_s.at[didx.at[0]], ssem
            ).wait()

        n = nchunk
        for j in range(NI):
            load_idx(j, j)
        for j in range(NB - 1):
            wait_idx()
            gath(j, j)
        wait_gath()
        scat(0, 0)
        wait_idx()
        gath((NB - 1) % NB, (NB - 1) % NI)
        wait_gath()
        scat(1 % NB, 1 % NI)

        def step(k, carry):
            wait_scat()
            load_idx(k + NI - 2, lax.rem(k + NI - 2, NI))
            wait_idx()
            gath(lax.rem(k + NB - 2, NB), lax.rem(k + NB - 2, NI))
            wait_gath()
            scat(lax.rem(k, NB), lax.rem(k, NI))
            return carry

        lax.fori_loop(2, n - NI + 2, step, 0)

        for k in range(n - NI + 2, n - NB + 2):
            wait_scat()
            wait_idx()
            gath((k + NB - 2) % NB, (k + NB - 2) % NI)
            wait_gath()
            scat(k % NB, k % NI)
        for k in range(n - NB + 2, n):
            wait_scat()
            wait_gath()
            scat(k % NB, k % NI)
        wait_scat()
        wait_scat()

        plsc.subcore_barrier()
        pltpu.sync_copy(
            agg_s.at[pl.ds(s * ROWS_PER_TILE, ROWS_PER_TILE)],
            out_hbm.at[pl.ds(c * NPAD + s * ROWS_PER_TILE, ROWS_PER_TILE)],
        )

    return scat_kernel(edgesx, feat)


_TCR = 1280


def _mm_body(a0_ref, a1_ref, c2_ref, ns_ref, w1_ref, b1_ref, w2_ref, g2_ref):
    hp = lax.Precision.HIGHEST
    w12 = jnp.dot(w1_ref[...], w2_ref[...],
                  preferred_element_type=jnp.float32, precision=hp)
    v = jnp.dot(b1_ref[...], w2_ref[...],
                preferred_element_type=jnp.float32, precision=hp)
    a = (a0_ref[...] + a1_ref[...]) * c2_ref[...]
    g2_ref[...] = (
        jnp.dot(a, w12, preferred_element_type=jnp.float32, precision=hp)
        + ns_ref[...] * v
    )


def _tc_matmuls(a0, a1, c2_col, ns_col, W1, b1r, W2p):
    row = lambda i: (i, 0)
    full = lambda i: (0, 0)
    return pl.pallas_call(
        _mm_body,
        grid=(NPAD // _TCR,),
        in_specs=[
            pl.BlockSpec((_TCR, D), row),
            pl.BlockSpec((_TCR, D), row),
            pl.BlockSpec((_TCR, 1), row),
            pl.BlockSpec((_TCR, 1), row),
            pl.BlockSpec((D, D), full),
            pl.BlockSpec((1, D), full),
            pl.BlockSpec((D, D2), full),
        ],
        out_specs=pl.BlockSpec((_TCR, D2), row),
        out_shape=jax.ShapeDtypeStruct((NPAD, D2), jnp.float32),
    )(a0, a1, c2_col, ns_col, W1, b1r, W2p)


_TCF = 2000


def _final_body(a0_ref, a1_ref, nd_ref, b2_ref, out_ref):
    out_ref[...] = (a0_ref[...] + a1_ref[...]) * nd_ref[...] + b2_ref[...]


def _tc_final(a0, a1, nd_col, b2r):
    row = lambda i: (i, 0)
    full = lambda i: (0, 0)
    return pl.pallas_call(
        _final_body,
        grid=(N // _TCF,),
        in_specs=[
            pl.BlockSpec((_TCF, D2), row),
            pl.BlockSpec((_TCF, D2), row),
            pl.BlockSpec((_TCF, 1), row),
            pl.BlockSpec((1, D2), full),
        ],
        out_specs=pl.BlockSpec((_TCF, D2), row),
        out_shape=jax.ShapeDtypeStruct((N, D2), jnp.float32),
    )(a0, a1, nd_col, b2r)


def kernel(x, edge_index, W1, b1, W2, b2):
    npadedge = EX - E - N
    loops = jnp.arange(N, dtype=jnp.int32)
    junk = N + jnp.arange(npadedge, dtype=jnp.int32) % (NPAD - N)
    srcx = jnp.concatenate([edge_index[0], loops, junk])
    dstx = jnp.concatenate([edge_index[1], loops, junk])
    edgesx = jnp.concatenate([srcx, dstx]).reshape(
        2 * NC * NS, EX // (NC * NS * CHUNK), CHUNK
    )

    norms, xs = _sc_degrees_norms_xs(edgesx, x)

    agg = _sc_scatter(edgesx, xs, D)
    c2_col = (norms[:NPAD] * norms[NPAD:])[:, None]
    g2 = _tc_matmuls(
        agg[:NPAD], agg[NPAD:], c2_col, norms[:NPAD, None],
        W1, b1[None, :], jnp.pad(W2, ((0, 0), (0, D2 - DOUT))),
    )

    agg2 = _sc_scatter(edgesx, g2, D2)
    out = _tc_final(
        agg2[:N], agg2[NPAD:NPAD + N], norms[NPAD:NPAD + N, None],
        jnp.pad(b2, (0, D2 - DOUT))[None, :],
    )
    return out[:, :DOUT]


# speedup vs baseline: 1.7069x; 1.0530x over previous
import functools

import jax
import jax.numpy as jnp
from jax import lax
from jax.experimental import pallas as pl
from jax.experimental.pallas import tpu as pltpu
from jax.experimental.pallas import tpu_sc as plsc

N = 10000
E = 320000
D = 128
DOUT = 40
D2 = 64
NPAD = 10240
NC = 2
NS = 16
CHUNK = 80
ROWS_PER_TILE = NPAD // NS

_MESH = dict(core_axis_name="c", subcore_axis_name="s")


def _sc_degrees(edges64):
    nblk = E // (NC * NS * CHUNK)

    @functools.partial(
        pl.kernel,
        out_type=jax.ShapeDtypeStruct((NC * NPAD,), jnp.float32),
        mesh=plsc.VectorSubcoreMesh(**_MESH),
        scratch_types=[
            pltpu.VMEM((2, nblk, CHUNK), jnp.int32),
            pltpu.VMEM((CHUNK,), jnp.float32),
            pltpu.VMEM((ROWS_PER_TILE,), jnp.float32),
            pltpu.SemaphoreType.DMA,
            pltpu.VMEM_SHARED((NPAD,), jnp.float32),
        ],
    )
    def deg_kernel(edges_hbm, out_hbm, idx_all, ones_v, zero_v, ssem, deg_s):
        c = lax.axis_index("c")
        s = lax.axis_index("s")
        w = c * NS + s

        def fill_zero(i, carry):
            zero_v[pl.ds(i * 16, 16)] = jnp.zeros((16,), jnp.float32)
            return carry

        lax.fori_loop(0, ROWS_PER_TILE // 16, fill_zero, 0)

        def fill_one(i, carry):
            ones_v[pl.ds(i * 16, 16)] = jnp.ones((16,), jnp.float32)
            return carry

        lax.fori_loop(0, CHUNK // 16, fill_one, 0)

        pltpu.sync_copy(edges_hbm.at[2 * w], idx_all.at[0])
        pltpu.sync_copy(edges_hbm.at[2 * w + 1], idx_all.at[1])
        pltpu.sync_copy(zero_v, deg_s.at[pl.ds(s * ROWS_PER_TILE, ROWS_PER_TILE)])
        plsc.subcore_barrier()

        for h in range(2):
            def step(j, carry):
                pltpu.async_copy(ones_v, deg_s.at[idx_all.at[h, j]], ssem, add=True)
                return carry

            lax.fori_loop(0, nblk, step, 0)

        def drain(j, carry):
            pltpu.make_async_copy(ones_v, deg_s.at[idx_all.at[0, 0]], ssem).wait()
            return carry

        lax.fori_loop(0, 2 * nblk, drain, 0)
        plsc.subcore_barrier()
        pltpu.sync_copy(
            deg_s.at[pl.ds(s * ROWS_PER_TILE, ROWS_PER_TILE)],
            out_hbm.at[pl.ds(c * NPAD + s * ROWS_PER_TILE, ROWS_PER_TILE)],
        )

    return deg_kernel(edges64)


def _sc_scatter(edges64, feat, dfeat):
    e_per_core = E // NC
    ept = e_per_core // NS
    csz = CHUNK
    NB = 3 if dfeat > 64 else 4
    nchunk = ept // csz
    NI = NB + 2
    ZR = 16

    @functools.partial(
        pl.kernel,
        out_type=jax.ShapeDtypeStruct((NC * NPAD, dfeat), jnp.float32),
        mesh=plsc.VectorSubcoreMesh(**_MESH),
        compiler_params=pltpu.CompilerParams(
            use_tc_tiling_on_sc=(dfeat % 128 == 0)
        ),
        scratch_types=[
            pltpu.VMEM((NI, csz), jnp.int32),
            pltpu.VMEM((NI, csz), jnp.int32),
            pltpu.VMEM((NB, csz, dfeat), jnp.float32),
            pltpu.VMEM((ZR, dfeat), jnp.float32),
            pltpu.SemaphoreType.DMA,
            pltpu.SemaphoreType.DMA,
            pltpu.SemaphoreType.DMA,
            pltpu.VMEM_SHARED((NPAD, dfeat), jnp.float32),
        ],
    )
    def scat_kernel(edges_hbm, feat_hbm, out_hbm, sidx, didx,
                    rows, zrows, isem, gsem, ssem, agg_s):
        c = lax.axis_index("c")
        s = lax.axis_index("s")
        wid = c * NS + s
        vpr = dfeat // 16

        def fill_zero(k, carry):
            zrows[k // vpr, pl.ds((k % vpr) * 16, 16)] = jnp.zeros((16,), jnp.float32)
            return carry

        lax.fori_loop(0, ZR * vpr, fill_zero, 0)

        def zero_chunk(k, carry):
            pltpu.sync_copy(
                zrows, agg_s.at[pl.ds(s * ROWS_PER_TILE + k * ZR, ZR)]
            )
            return carry

        lax.fori_loop(0, ROWS_PER_TILE // ZR, zero_chunk, 0)
        plsc.subcore_barrier()

        def load_idx(j, slot):
            pltpu.async_copy(edges_hbm.at[wid, j], sidx.at[slot], isem)
            pltpu.async_copy(edges_hbm.at[NC * NS + wid, j], didx.at[slot], isem)

        def wait_idx():
            pltpu.make_async_copy(edges_hbm.at[0, 0], sidx.at[0], isem).wait()
            pltpu.make_async_copy(edges_hbm.at[0, 0], didx.at[0], isem).wait()

        def gath(b, slot):
            pltpu.async_copy(feat_hbm.at[sidx.at[slot]], rows.at[b], gsem)

        def wait_gath():
            pltpu.make_async_copy(
                feat_hbm.at[sidx.at[0]], rows.at[0], gsem
            ).wait()

        def scat(b, slot):
            pltpu.async_copy(rows.at[b], agg_s.at[didx.at[slot]], ssem, add=True)

        def wait_scat():
            pltpu.make_async_copy(
                rows.at[0], agg_s.at[didx.at[0]], ssem
            ).wait()

        n = nchunk
        for j in range(NI):
            load_idx(j, j)
        for j in range(NB - 1):
            wait_idx()
            gath(j, j)
        wait_gath()
        scat(0, 0)
        wait_idx()
        gath((NB - 1) % NB, (NB - 1) % NI)
        wait_gath()
        scat(1 % NB, 1 % NI)

        def step(k, carry):
            wait_scat()
            load_idx(k + NI - 2, lax.rem(k + NI - 2, NI))
            wait_idx()
            gath(lax.rem(k + NB - 2, NB), lax.rem(k + NB - 2, NI))
            wait_gath()
            scat(lax.rem(k, NB), lax.rem(k, NI))
            return carry

        lax.fori_loop(2, n - NI + 2, step, 0)

        for k in range(n - NI + 2, n - NB + 2):
            wait_scat()
            wait_idx()
            gath((k + NB - 2) % NB, (k + NB - 2) % NI)
            wait_gath()
            scat(k % NB, k % NI)
        for k in range(n - NB + 2, n):
            wait_scat()
            wait_gath()
            scat(k % NB, k % NI)
        wait_scat()
        wait_scat()

        plsc.subcore_barrier()
        pltpu.sync_copy(
            agg_s.at[pl.ds(s * ROWS_PER_TILE, ROWS_PER_TILE)],
            out_hbm.at[pl.ds(c * NPAD + s * ROWS_PER_TILE, ROWS_PER_TILE)],
        )

    return scat_kernel(edges64, feat)


_TCR = 2000


def _norm_body(x_ref, do_ref, di_ref, xs_ref, ns_ref, nd_ref):
    ns = lax.rsqrt(do_ref[...] + 1.0)
    nd = lax.rsqrt(di_ref[...] + 1.0)
    xs_ref[...] = x_ref[...] * ns
    ns_ref[...] = ns
    nd_ref[...] = nd


def _tc_norm(x, do_col, di_col):
    row = lambda i: (i, 0)
    return pl.pallas_call(
        _norm_body,
        grid=(N // _TCR,),
        in_specs=[
            pl.BlockSpec((_TCR, D), row),
            pl.BlockSpec((_TCR, 1), row),
            pl.BlockSpec((_TCR, 1), row),
        ],
        out_specs=[
            pl.BlockSpec((_TCR, D), row),
            pl.BlockSpec((_TCR, 1), row),
            pl.BlockSpec((_TCR, 1), row),
        ],
        out_shape=[
            jax.ShapeDtypeStruct((N, D), jnp.float32),
            jax.ShapeDtypeStruct((N, 1), jnp.float32),
            jax.ShapeDtypeStruct((N, 1), jnp.float32),
        ],
    )(x, do_col, di_col)


def _mm_body(a0_ref, a1_ref, xs_ref, nd_ref, ns_ref, w1_ref, b1_ref, w2_ref, g2_ref):
    a = (a0_ref[...] + a1_ref[...] + xs_ref[...]) * nd_ref[...]
    h1 = (
        jnp.dot(a, w1_ref[...], preferred_element_type=jnp.float32,
                precision=lax.Precision.HIGHEST)
        + b1_ref[...]
    )
    g2_ref[...] = jnp.dot(
        h1 * ns_ref[...], w2_ref[...], preferred_element_type=jnp.float32,
        precision=lax.Precision.HIGHEST,
    )


def _tc_matmuls(a0, a1, xs, nd_col, ns_col, W1, b1r, W2p):
    row = lambda i: (i, 0)
    full = lambda i: (0, 0)
    return pl.pallas_call(
        _mm_body,
        grid=(N // _TCR,),
        in_specs=[
            pl.BlockSpec((_TCR, D), row),
            pl.BlockSpec((_TCR, D), row),
            pl.BlockSpec((_TCR, D), row),
            pl.BlockSpec((_TCR, 1), row),
            pl.BlockSpec((_TCR, 1), row),
            pl.BlockSpec((D, D), full),
            pl.BlockSpec((1, D), full),
            pl.BlockSpec((D, D2), full),
        ],
        out_specs=pl.BlockSpec((_TCR, D2), row),
        out_shape=jax.ShapeDtypeStruct((N, D2), jnp.float32),
    )(a0, a1, xs, nd_col, ns_col, W1, b1r, W2p)


def _final_body(a0_ref, a1_ref, g2_ref, nd_ref, b2_ref, out_ref):
    out_ref[...] = (
        (a0_ref[...] + a1_ref[...] + g2_ref[...]) * nd_ref[...] + b2_ref[...]
    )


_TCP = 1000


def _tc_final(a0p, a1p, g2p, ndp, b2p):
    row = lambda i: (i, 0)
    full = lambda i: (0, 0)
    return pl.pallas_call(
        _final_body,
        grid=(N // 2 // _TCP,),
        in_specs=[
            pl.BlockSpec((_TCP, 2 * D2), row),
            pl.BlockSpec((_TCP, 2 * D2), row),
            pl.BlockSpec((_TCP, 2 * D2), row),
            pl.BlockSpec((_TCP, 2 * D2), row),
            pl.BlockSpec((1, 2 * D2), full),
        ],
        out_specs=pl.BlockSpec((_TCP, 2 * D2), row),
        out_shape=jax.ShapeDtypeStruct((N // 2, 2 * D2), jnp.float32),
    )(a0p, a1p, g2p, ndp, b2p)


def kernel(x, edge_index, W1, b1, W2, b2):
    nblk = E // (NC * NS * CHUNK)
    edges64 = edge_index.reshape(2 * NC * NS, nblk, CHUNK)

    deg = _sc_degrees(edges64)
    do_col = deg[:N, None]
    di_col = deg[NPAD:NPAD + N, None]

    xs, ns_col, nd_col = _tc_norm(x, do_col, di_col)

    agg = _sc_scatter(edges64, xs, D)
    g2 = _tc_matmuls(
        agg[:N], agg[NPAD:NPAD + N], xs, nd_col, ns_col,
        W1, b1[None, :], jnp.pad(W2, ((0, 0), (0, D2 - DOUT))),
    )

    agg2 = _sc_scatter(edges64, g2, D2)
    a2p = agg2.reshape(NC, NPAD // 2, 2 * D2)
    g2p = g2.reshape(N // 2, 2 * D2)
    ndp = jnp.broadcast_to(
        nd_col.reshape(N // 2, 2, 1), (N // 2, 2, D2)
    ).reshape(N // 2, 2 * D2)
    b2p = jnp.tile(jnp.pad(b2, (0, D2 - DOUT)), 2)[None, :]
    outp = _tc_final(a2p[0, : N // 2], a2p[1, : N // 2], g2p, ndp, b2p)
    return outp.reshape(N, D2)[:, :DOUT]
